# Initial kernel scaffold; baseline (speedup 1.0000x reference)
import functools

import jax
import jax.numpy as jnp
from jax import lax
from jax.experimental import pallas as pl
from jax.experimental.pallas import tpu as pltpu
from jax.experimental.pallas import tpu_sc as plsc

N = 10000
D = 128
E = 320000
B = 16384

NC = 2
NS = 16
L = 16
NT = NC * NS

NPAD = 10240
CK = 128
EPAD = 327680
EPT = EPAD // NT
ROWS_PT = NPAD // NS

_mesh = plsc.VectorSubcoreMesh(
    core_axis_name="c", subcore_axis_name="s", num_cores=NC, num_subcores=NS
)


def _mm_body(emb_ref, w_ref, b_ref, out_ref):
    out_ref[...] = (
        jnp.dot(emb_ref[...], w_ref[...], preferred_element_type=jnp.float32)
        + b_ref[...]
    )


def _matmul(embed, W, b):
    return pl.pallas_call(
        _mm_body,
        grid=(25,),
        in_specs=[
            pl.BlockSpec((400, D), lambda i: (i, 0)),
            pl.BlockSpec((D, D), lambda i: (0, 0)),
            pl.BlockSpec((1, D), lambda i: (0, 0)),
        ],
        out_specs=pl.BlockSpec((400, D), lambda i: (i, 0)),
        out_shape=jax.ShapeDtypeStruct((N, D), jnp.float32),
    )(embed, W, b.reshape(1, D))


@functools.partial(
    pl.kernel,
    out_type=jax.ShapeDtypeStruct((NC * NPAD, D), jnp.float32),
    mesh=_mesh,
    scratch_types=[
        pltpu.VMEM((CK,), jnp.int32),
        pltpu.VMEM((CK,), jnp.int32),
        pltpu.VMEM((CK,), jnp.float32),
        pltpu.VMEM((CK, D), jnp.float32),
        pltpu.VMEM_SHARED((NPAD, D), jnp.float32),
        pltpu.SemaphoreType.DMA,
    ],
)
def _aggregate(h_hbm, col_hbm, row_hbm, val_hbm, out_hbm,
               col_v, row_v, val_v, gat_v, acc_sh, sem):
    c = lax.axis_index("c")
    s = lax.axis_index("s")
    wid = c * NS + s

    zero = jnp.zeros((L,), jnp.float32)

    def _zrow(i, carry):
        for q in range(D // L):
            gat_v[i, pl.ds(q * L, L)] = zero
        return carry

    lax.fori_loop(0, CK, _zrow, 0)
    base_row = s * ROWS_PT
    for k in range(ROWS_PT // CK):
        pltpu.sync_copy(gat_v, acc_sh.at[pl.ds(base_row + k * CK, CK)])
    plsc.subcore_barrier()

    ebase = wid * EPT

    def _chunk(j, carry):
        off = ebase + j * CK
        pltpu.sync_copy(col_hbm.at[pl.ds(off, CK)], col_v)
        pltpu.sync_copy(row_hbm.at[pl.ds(off, CK)], row_v)
        pltpu.sync_copy(val_hbm.at[pl.ds(off, CK)], val_v)
        pltpu.async_copy(h_hbm.at[col_v], gat_v, sem).wait()

        def _edge(r, carry2):
            sv = plsc.load_gather(val_v, [jnp.full((L,), r, jnp.int32)])
            for q in range(D // L):
                gat_v[r, pl.ds(q * L, L)] = gat_v[r, pl.ds(q * L, L)] * sv
            return carry2

        lax.fori_loop(0, CK, _edge, 0)
        pltpu.sync_copy(gat_v, acc_sh.at[row_v], add=True)
        return carry

    lax.fori_loop(0, EPT // CK, _chunk, 0)
    plsc.subcore_barrier()

    for k in range(ROWS_PT // CK):
        r0 = base_row + k * CK
        pltpu.sync_copy(acc_sh.at[pl.ds(r0, CK)], gat_v)
        pltpu.sync_copy(gat_v, out_hbm.at[pl.ds(c * NPAD + r0, CK)])


_LN_BLK = 256


def _ln_body(p_ref, g_ref, be_ref, o_ref):
    i = pl.program_id(0)
    h = jnp.maximum(p_ref[0] + p_ref[1], 0.0)
    mu = jnp.mean(h, axis=-1, keepdims=True)
    d = h - mu
    var = jnp.mean(d * d, axis=-1, keepdims=True)
    y = d * lax.rsqrt(var + 1e-5) * g_ref[...] + be_ref[...]
    rows = i * _LN_BLK + lax.broadcasted_iota(jnp.int32, (_LN_BLK, 1), 0)
    o_ref[...] = jnp.where(rows < N, y, 0.0)


def _layernorm(partials, gamma, beta):
    return pl.pallas_call(
        _ln_body,
        grid=(NPAD // _LN_BLK,),
        in_specs=[
            pl.BlockSpec((NC, _LN_BLK, D), lambda i: (0, i, 0)),
            pl.BlockSpec((1, D), lambda i: (0, 0)),
            pl.BlockSpec((1, D), lambda i: (0, 0)),
        ],
        out_specs=pl.BlockSpec((_LN_BLK, D), lambda i: (i, 0)),
        out_shape=jax.ShapeDtypeStruct((NPAD, D), jnp.float32),
    )(partials.reshape(NC, NPAD, D), gamma.reshape(1, D), beta.reshape(1, D))


BPT = B // NT

@functools.partial(
    pl.kernel,
    out_type=jax.ShapeDtypeStruct((B, D), jnp.float32),
    mesh=_mesh,
    scratch_types=[
        pltpu.VMEM((CK,), jnp.int32),
        pltpu.VMEM((CK, D), jnp.float32),
        pltpu.SemaphoreType.DMA,
    ],
)
def _lookup(hn_hbm, x_hbm, out_hbm, idx_v, rows_v, sem):
    c = lax.axis_index("c")
    s = lax.axis_index("s")
    wid = c * NS + s
    base = wid * BPT

    def _chunk(j, carry):
        off = base + j * CK
        pltpu.sync_copy(x_hbm.at[pl.ds(off, CK)], idx_v)

        def _grp(g, carry2):
            xx = idx_v[pl.ds(g * L, L)]
            ok = (xx >= 1) & (xx < N + 1)
            idx_v[pl.ds(g * L, L)] = jnp.where(ok, xx - 1, N)
            return carry2

        lax.fori_loop(0, CK // L, _grp, 0)
        pltpu.async_copy(hn_hbm.at[idx_v], rows_v, sem).wait()
        pltpu.sync_copy(rows_v, out_hbm.at[pl.ds(off, CK)])
        return carry

    lax.fori_loop(0, BPT // CK, _chunk, 0)


def kernel(x, embed, W, b, edge_row, edge_col, edge_val, ln_gamma, ln_beta):
    h = _matmul(embed.astype(jnp.float32), W, b)
    pad = EPAD - E
    col_p = jnp.pad(edge_col, (0, pad))
    row_p = jnp.pad(edge_row, (0, pad))
    val_p = jnp.pad(edge_val, (0, pad))
    partials = _aggregate(h, col_p, row_p, val_p)
    hn = _layernorm(partials, ln_gamma, ln_beta)
    final = _lookup(hn, x)
    recon_loss = jnp.zeros((1,), jnp.float32)
    return (final, recon_loss)


# speedup vs baseline: 2.7393x; 2.7393x over previous
import functools

import jax
import jax.numpy as jnp
from jax import lax
from jax.experimental import pallas as pl
from jax.experimental.pallas import tpu as pltpu
from jax.experimental.pallas import tpu_sc as plsc

N = 10000
D = 128
E = 320000
B = 16384

NC = 2
NS = 16
L = 16
NT = NC * NS

NPAD = 10240
CK = 128
EPAD = 327680
EPT = EPAD // NT
ROWS_PT = NPAD // NS

def _splat(vec, e):
    return lax.gather(
        vec,
        jnp.full((L, 1), e, jnp.int32),
        dimension_numbers=lax.GatherDimensionNumbers(
            offset_dims=(), collapsed_slice_dims=(0,), start_index_map=(0,)),
        slice_sizes=(1,),
        mode=lax.GatherScatterMode.PROMISE_IN_BOUNDS,
    )


_mesh = plsc.VectorSubcoreMesh(
    core_axis_name="c", subcore_axis_name="s", num_cores=NC, num_subcores=NS
)


def _mm_body(emb_ref, w_ref, b_ref, out_ref):
    out_ref[...] = (
        jnp.dot(emb_ref[...], w_ref[...], preferred_element_type=jnp.float32)
        + b_ref[...]
    )


def _matmul(embed, W, b):
    return pl.pallas_call(
        _mm_body,
        grid=(25,),
        in_specs=[
            pl.BlockSpec((400, D), lambda i: (i, 0)),
            pl.BlockSpec((D, D), lambda i: (0, 0)),
            pl.BlockSpec((1, D), lambda i: (0, 0)),
        ],
        out_specs=pl.BlockSpec((400, D), lambda i: (i, 0)),
        out_shape=jax.ShapeDtypeStruct((N, D), jnp.float32),
    )(embed, W, b.reshape(1, D))


@functools.partial(
    pl.kernel,
    out_type=jax.ShapeDtypeStruct((NC * NPAD, D), jnp.float32),
    mesh=_mesh,
    scratch_types=[
        pltpu.VMEM((CK,), jnp.int32),
        pltpu.VMEM((CK,), jnp.int32),
        pltpu.VMEM((CK,), jnp.float32),
        pltpu.VMEM((CK, D), jnp.float32),
        pltpu.VMEM_SHARED((NPAD, D), jnp.float32),
        pltpu.SemaphoreType.DMA,
    ],
)
def _aggregate(h_hbm, col_hbm, row_hbm, val_hbm, out_hbm,
               col_v, row_v, val_v, gat_v, acc_sh, sem):
    c = lax.axis_index("c")
    s = lax.axis_index("s")
    wid = c * NS + s

    zero = jnp.zeros((L,), jnp.float32)

    def _zrow(i, carry):
        for q in range(D // L):
            gat_v[i, pl.ds(q * L, L)] = zero
        return carry

    lax.fori_loop(0, CK, _zrow, 0)
    base_row = s * ROWS_PT
    for k in range(ROWS_PT // CK):
        pltpu.sync_copy(gat_v, acc_sh.at[pl.ds(base_row + k * CK, CK)])
    plsc.subcore_barrier()

    ebase = wid * EPT

    def _chunk(j, carry):
        off = ebase + j * CK
        pltpu.sync_copy(col_hbm.at[pl.ds(off, CK)], col_v)
        pltpu.sync_copy(row_hbm.at[pl.ds(off, CK)], row_v)
        pltpu.sync_copy(val_hbm.at[pl.ds(off, CK)], val_v)
        pltpu.async_copy(h_hbm.at[col_v], gat_v, sem).wait()

        def _grp(g, carry2):
            vv = val_v[pl.ds(g * L, L)]
            for e in range(L):
                sv = _splat(vv, e)
                r = g * L + e
                for q in range(D // L):
                    gat_v[r, pl.ds(q * L, L)] = gat_v[r, pl.ds(q * L, L)] * sv
            return carry2

        lax.fori_loop(0, CK // L, _grp, 0)
        pltpu.sync_copy(gat_v, acc_sh.at[row_v], add=True)
        return carry

    lax.fori_loop(0, EPT // CK, _chunk, 0)
    plsc.subcore_barrier()

    for k in range(ROWS_PT // CK):
        r0 = base_row + k * CK
        pltpu.sync_copy(acc_sh.at[pl.ds(r0, CK)], gat_v)
        pltpu.sync_copy(gat_v, out_hbm.at[pl.ds(c * NPAD + r0, CK)])


_LN_BLK = 256


def _ln_body(p_ref, g_ref, be_ref, o_ref):
    i = pl.program_id(0)
    h = jnp.maximum(p_ref[0] + p_ref[1], 0.0)
    mu = jnp.mean(h, axis=-1, keepdims=True)
    d = h - mu
    var = jnp.mean(d * d, axis=-1, keepdims=True)
    y = d * lax.rsqrt(var + 1e-5) * g_ref[...] + be_ref[...]
    rows = i * _LN_BLK + lax.broadcasted_iota(jnp.int32, (_LN_BLK, 1), 0)
    o_ref[...] = jnp.where(rows < N, y, 0.0)


def _layernorm(partials, gamma, beta):
    return pl.pallas_call(
        _ln_body,
        grid=(NPAD // _LN_BLK,),
        in_specs=[
            pl.BlockSpec((NC, _LN_BLK, D), lambda i: (0, i, 0)),
            pl.BlockSpec((1, D), lambda i: (0, 0)),
            pl.BlockSpec((1, D), lambda i: (0, 0)),
        ],
        out_specs=pl.BlockSpec((_LN_BLK, D), lambda i: (i, 0)),
        out_shape=jax.ShapeDtypeStruct((NPAD, D), jnp.float32),
    )(partials.reshape(NC, NPAD, D), gamma.reshape(1, D), beta.reshape(1, D))


BPT = B // NT

@functools.partial(
    pl.kernel,
    out_type=jax.ShapeDtypeStruct((B, D), jnp.float32),
    mesh=_mesh,
    scratch_types=[
        pltpu.VMEM((CK,), jnp.int32),
        pltpu.VMEM((CK, D), jnp.float32),
        pltpu.SemaphoreType.DMA,
    ],
)
def _lookup(hn_hbm, x_hbm, out_hbm, idx_v, rows_v, sem):
    c = lax.axis_index("c")
    s = lax.axis_index("s")
    wid = c * NS + s
    base = wid * BPT

    def _chunk(j, carry):
        off = base + j * CK
        pltpu.sync_copy(x_hbm.at[pl.ds(off, CK)], idx_v)

        def _grp(g, carry2):
            xx = idx_v[pl.ds(g * L, L)]
            ok = (xx >= 1) & (xx < N + 1)
            idx_v[pl.ds(g * L, L)] = jnp.where(ok, xx - 1, N)
            return carry2

        lax.fori_loop(0, CK // L, _grp, 0)
        pltpu.async_copy(hn_hbm.at[idx_v], rows_v, sem).wait()
        pltpu.sync_copy(rows_v, out_hbm.at[pl.ds(off, CK)])
        return carry

    lax.fori_loop(0, BPT // CK, _chunk, 0)


def kernel(x, embed, W, b, edge_row, edge_col, edge_val, ln_gamma, ln_beta):
    h = _matmul(embed.astype(jnp.float32), W, b)
    pad = EPAD - E
    col_p = jnp.pad(edge_col, (0, pad))
    row_p = jnp.pad(edge_row, (0, pad))
    val_p = jnp.pad(edge_val, (0, pad))
    partials = _aggregate(h, col_p, row_p, val_p)
    hn = _layernorm(partials, ln_gamma, ln_beta)
    final = _lookup(hn, x)
    recon_loss = jnp.zeros((1,), jnp.float32)
    return (final, recon_loss)


# speedup vs baseline: 6.2422x; 2.2787x over previous
import functools

import jax
import jax.numpy as jnp
from jax import lax
from jax.experimental import pallas as pl
from jax.experimental.pallas import tpu as pltpu
from jax.experimental.pallas import tpu_sc as plsc

N = 10000
D = 128
E = 320000
B = 16384

NC = 2
NS = 16
L = 16
NT = NC * NS

NPAD = 10240
CK = 112
NCH = 90
EPT = NCH * CK
EPAD = NT * EPT
ROWS_PT = NPAD // NS

def _splat(vec, e):
    return lax.gather(
        vec,
        jnp.full((L, 1), e, jnp.int32),
        dimension_numbers=lax.GatherDimensionNumbers(
            offset_dims=(), collapsed_slice_dims=(0,), start_index_map=(0,)),
        slice_sizes=(1,),
        mode=lax.GatherScatterMode.PROMISE_IN_BOUNDS,
    )


_mesh = plsc.VectorSubcoreMesh(
    core_axis_name="c", subcore_axis_name="s", num_cores=NC, num_subcores=NS
)


def _mm_body(emb_ref, w_ref, b_ref, out_ref):
    out_ref[...] = (
        jnp.dot(emb_ref[...], w_ref[...], preferred_element_type=jnp.float32)
        + b_ref[...]
    )


def _matmul(embed, W, b):
    return pl.pallas_call(
        _mm_body,
        grid=(25,),
        in_specs=[
            pl.BlockSpec((400, D), lambda i: (i, 0)),
            pl.BlockSpec((D, D), lambda i: (0, 0)),
            pl.BlockSpec((1, D), lambda i: (0, 0)),
        ],
        out_specs=pl.BlockSpec((400, D), lambda i: (i, 0)),
        out_shape=jax.ShapeDtypeStruct((N, D), jnp.float32),
    )(embed, W, b.reshape(1, D))


@functools.partial(
    pl.kernel,
    out_type=jax.ShapeDtypeStruct((NC * NPAD, D), jnp.float32),
    mesh=_mesh,
    scratch_types=[
        [pltpu.VMEM((CK,), jnp.int32),
         pltpu.VMEM((CK,), jnp.int32),
         pltpu.VMEM((CK,), jnp.float32)],
        [pltpu.VMEM((CK,), jnp.int32),
         pltpu.VMEM((CK,), jnp.int32),
         pltpu.VMEM((CK,), jnp.float32)],
        [pltpu.VMEM((CK,), jnp.int32),
         pltpu.VMEM((CK,), jnp.int32),
         pltpu.VMEM((CK,), jnp.float32)],
        pltpu.VMEM((CK, D), jnp.float32),
        pltpu.VMEM((CK, D), jnp.float32),
        pltpu.VMEM((CK, D), jnp.float32),
        pltpu.VMEM_SHARED((NPAD, D), jnp.float32),
        pltpu.SemaphoreType.DMA,
        pltpu.SemaphoreType.DMA,
        pltpu.SemaphoreType.DMA,
        pltpu.SemaphoreType.DMA,
        pltpu.SemaphoreType.DMA,
        pltpu.SemaphoreType.DMA,
        pltpu.SemaphoreType.DMA,
        pltpu.SemaphoreType.DMA,
        pltpu.SemaphoreType.DMA,
    ],
)
def _aggregate(h_hbm, col_hbm, row_hbm, val_hbm, out_hbm,
               pk0, pk1, pk2, gat0, gat1, gat2, acc_sh,
               si0, si1, si2, sg0, sg1, sg2, ss0, ss1, ss2):
    c = lax.axis_index("c")
    s = lax.axis_index("s")
    wid = c * NS + s
    pk = (pk0, pk1, pk2)
    gat = (gat0, gat1, gat2)
    sem_idx = (si0, si1, si2)
    sem_gat = (sg0, sg1, sg2)
    sem_sc = (ss0, ss1, ss2)

    zero = jnp.zeros((L,), jnp.float32)

    def _zrow(i, carry):
        for q in range(D // L):
            gat0[i, pl.ds(q * L, L)] = zero
        return carry

    lax.fori_loop(0, CK, _zrow, 0)
    base_row = s * ROWS_PT
    slabs = [(k * CK, CK) for k in range(ROWS_PT // CK)]
    if ROWS_PT % CK:
        slabs.append(((ROWS_PT // CK) * CK, ROWS_PT % CK))
    for (r0, nr) in slabs:
        pltpu.sync_copy(gat0.at[pl.ds(0, nr)],
                        acc_sh.at[pl.ds(base_row + r0, nr)])
    plsc.subcore_barrier()

    cbase = wid * NCH

    def _issue_idx(j, p):
        off = (cbase + j) * CK
        colv, rowv, valv = pk[p]
        pltpu.async_copy(col_hbm.at[pl.ds(off, CK)], colv, sem_idx[p])
        pltpu.async_copy(row_hbm.at[pl.ds(off, CK)], rowv, sem_idx[p])
        pltpu.async_copy(val_hbm.at[pl.ds(off, CK)], valv, sem_idx[p])

    def _wait_idx(p):
        colv, rowv, valv = pk[p]
        pltpu.make_async_copy(col_hbm.at[pl.ds(0, CK)], colv,
                              sem_idx[p]).wait()
        pltpu.make_async_copy(row_hbm.at[pl.ds(0, CK)], rowv,
                              sem_idx[p]).wait()
        pltpu.make_async_copy(val_hbm.at[pl.ds(0, CK)], valv,
                              sem_idx[p]).wait()

    def _issue_gat(p):
        pltpu.async_copy(h_hbm.at[pk[p][0]], gat[p], sem_gat[p])

    def _wait_gat(p):
        pltpu.make_async_copy(h_hbm.at[pk[p][0]], gat[p],
                              sem_gat[p]).wait()

    def _issue_sc(p):
        pltpu.make_async_copy(gat[p], acc_sh.at[pk[p][1]],
                              sem_sc[p]).start(add=True)

    def _wait_sc(p):
        pltpu.make_async_copy(gat[p], acc_sh.at[pk[p][1]],
                              sem_sc[p]).wait()

    def _scale(p):
        valv = pk[p][2]
        gp = gat[p]

        def _grp(g, carry2):
            vv = valv[pl.ds(g * L, L)]
            for e in range(L):
                sv = _splat(vv, e)
                r = g * L + e
                for q in range(D // L):
                    gp[r, pl.ds(q * L, L)] = gp[r, pl.ds(q * L, L)] * sv
            return carry2

        lax.fori_loop(0, CK // L, _grp, 0)

    def _step(j, p, pn, pp):
        _wait_gat(p)

        @pl.when(j >= 1)
        def _():
            _wait_sc(pp)

        @pl.when(j + 2 < NCH)
        def _():
            _issue_idx(j + 2, pp)

        @pl.when(j + 1 < NCH)
        def _():
            _wait_idx(pn)
            _issue_gat(pn)

        _scale(p)
        _issue_sc(p)

    _issue_idx(0, 0)
    _wait_idx(0)
    _issue_gat(0)
    _issue_idx(1, 1)

    def _trip(t, carry):
        j = 3 * t
        _step(j, 0, 1, 2)
        _step(j + 1, 1, 2, 0)
        _step(j + 2, 2, 0, 1)
        return carry

    lax.fori_loop(0, NCH // 3, _trip, 0)
    _wait_sc((NCH - 1) % 3)
    plsc.subcore_barrier()

    for (r0, nr) in slabs:
        pltpu.sync_copy(acc_sh.at[pl.ds(base_row + r0, nr)],
                        gat0.at[pl.ds(0, nr)])
        pltpu.sync_copy(gat0.at[pl.ds(0, nr)],
                        out_hbm.at[pl.ds(c * NPAD + base_row + r0, nr)])


_LN_BLK = 256


def _ln_body(p_ref, g_ref, be_ref, o_ref):
    i = pl.program_id(0)
    h = jnp.maximum(p_ref[0] + p_ref[1], 0.0)
    mu = jnp.mean(h, axis=-1, keepdims=True)
    d = h - mu
    var = jnp.mean(d * d, axis=-1, keepdims=True)
    y = d * lax.rsqrt(var + 1e-5) * g_ref[...] + be_ref[...]
    rows = i * _LN_BLK + lax.broadcasted_iota(jnp.int32, (_LN_BLK, 1), 0)
    o_ref[...] = jnp.where(rows < N, y, 0.0)


def _layernorm(partials, gamma, beta):
    return pl.pallas_call(
        _ln_body,
        grid=(NPAD // _LN_BLK,),
        in_specs=[
            pl.BlockSpec((NC, _LN_BLK, D), lambda i: (0, i, 0)),
            pl.BlockSpec((1, D), lambda i: (0, 0)),
            pl.BlockSpec((1, D), lambda i: (0, 0)),
        ],
        out_specs=pl.BlockSpec((_LN_BLK, D), lambda i: (i, 0)),
        out_shape=jax.ShapeDtypeStruct((NPAD, D), jnp.float32),
    )(partials.reshape(NC, NPAD, D), gamma.reshape(1, D), beta.reshape(1, D))


BPT = B // NT
CKD = 128

@functools.partial(
    pl.kernel,
    out_type=jax.ShapeDtypeStruct((B, D), jnp.float32),
    mesh=_mesh,
    scratch_types=[
        pltpu.VMEM((CKD,), jnp.int32),
        pltpu.VMEM((CKD, D), jnp.float32),
        pltpu.SemaphoreType.DMA,
    ],
)
def _lookup(hn_hbm, x_hbm, out_hbm, idx_v, rows_v, sem):
    c = lax.axis_index("c")
    s = lax.axis_index("s")
    wid = c * NS + s
    base = wid * BPT

    def _chunk(j, carry):
        off = base + j * CKD
        pltpu.sync_copy(x_hbm.at[pl.ds(off, CKD)], idx_v)

        def _grp(g, carry2):
            xx = idx_v[pl.ds(g * L, L)]
            ok = (xx >= 1) & (xx < N + 1)
            idx_v[pl.ds(g * L, L)] = jnp.where(ok, xx - 1, N)
            return carry2

        lax.fori_loop(0, CKD // L, _grp, 0)
        pltpu.async_copy(hn_hbm.at[idx_v], rows_v, sem).wait()
        pltpu.sync_copy(rows_v, out_hbm.at[pl.ds(off, CKD)])
        return carry

    lax.fori_loop(0, BPT // CKD, _chunk, 0)


def kernel(x, embed, W, b, edge_row, edge_col, edge_val, ln_gamma, ln_beta):
    h = _matmul(embed.astype(jnp.float32), W, b)
    pad = EPAD - E
    col_p = jnp.pad(edge_col, (0, pad))
    row_p = jnp.pad(edge_row, (0, pad))
    val_p = jnp.pad(edge_val, (0, pad))
    partials = _aggregate(h, col_p, row_p, val_p)
    hn = _layernorm(partials, ln_gamma, ln_beta)
    final = _lookup(hn, x)
    recon_loss = jnp.zeros((1,), jnp.float32)
    return (final, recon_loss)


# speedup vs baseline: 6.2610x; 1.0030x over previous
import functools

import jax
import jax.numpy as jnp
from jax import lax
from jax.experimental import pallas as pl
from jax.experimental.pallas import tpu as pltpu
from jax.experimental.pallas import tpu_sc as plsc

N = 10000
D = 128
E = 320000
B = 16384

NC = 2
NS = 16
L = 16
NT = NC * NS

NPAD = 10240
CK = 112
NCH = 90
EPT = NCH * CK
EPAD = NT * EPT
ROWS_PT = NPAD // NS

def _splat(vec, e):
    return lax.gather(
        vec,
        jnp.full((L, 1), e, jnp.int32),
        dimension_numbers=lax.GatherDimensionNumbers(
            offset_dims=(), collapsed_slice_dims=(0,), start_index_map=(0,)),
        slice_sizes=(1,),
        mode=lax.GatherScatterMode.PROMISE_IN_BOUNDS,
    )


_mesh = plsc.VectorSubcoreMesh(
    core_axis_name="c", subcore_axis_name="s", num_cores=NC, num_subcores=NS
)


def _mm_body(emb_ref, w_ref, b_ref, out_ref):
    out_ref[...] = (
        jnp.dot(emb_ref[...], w_ref[...], preferred_element_type=jnp.float32)
        + b_ref[...]
    )


def _matmul(embed, W, b):
    return pl.pallas_call(
        _mm_body,
        grid=(25,),
        in_specs=[
            pl.BlockSpec((400, D), lambda i: (i, 0)),
            pl.BlockSpec((D, D), lambda i: (0, 0)),
            pl.BlockSpec((1, D), lambda i: (0, 0)),
        ],
        out_specs=pl.BlockSpec((400, D), lambda i: (i, 0)),
        out_shape=jax.ShapeDtypeStruct((N, D), jnp.float32),
    )(embed, W, b.reshape(1, D))


@functools.partial(
    pl.kernel,
    out_type=jax.ShapeDtypeStruct((NC * NPAD, D), jnp.float32),
    mesh=_mesh,
    scratch_types=[
        [pltpu.VMEM((2, CK), jnp.int32),
         pltpu.VMEM((CK,), jnp.float32)],
        [pltpu.VMEM((2, CK), jnp.int32),
         pltpu.VMEM((CK,), jnp.float32)],
        [pltpu.VMEM((2, CK), jnp.int32),
         pltpu.VMEM((CK,), jnp.float32)],
        pltpu.VMEM((CK, D), jnp.float32),
        pltpu.VMEM((CK, D), jnp.float32),
        pltpu.VMEM((CK, D), jnp.float32),
        pltpu.VMEM_SHARED((NPAD, D), jnp.float32),
        pltpu.SemaphoreType.DMA,
        pltpu.SemaphoreType.DMA,
        pltpu.SemaphoreType.DMA,
        pltpu.SemaphoreType.DMA,
        pltpu.SemaphoreType.DMA,
        pltpu.SemaphoreType.DMA,
        pltpu.SemaphoreType.DMA,
        pltpu.SemaphoreType.DMA,
        pltpu.SemaphoreType.DMA,
    ],
)
def _aggregate(h_hbm, pk_hbm, val_hbm, out_hbm,
               pk0, pk1, pk2, gat0, gat1, gat2, acc_sh,
               si0, si1, si2, sg0, sg1, sg2, ss0, ss1, ss2):
    c = lax.axis_index("c")
    s = lax.axis_index("s")
    wid = c * NS + s
    pk = (pk0, pk1, pk2)
    gat = (gat0, gat1, gat2)
    sem_idx = (si0, si1, si2)
    sem_gat = (sg0, sg1, sg2)
    sem_sc = (ss0, ss1, ss2)

    zero = jnp.zeros((L,), jnp.float32)

    def _zrow(i, carry):
        for q in range(D // L):
            gat0[i, pl.ds(q * L, L)] = zero
        return carry

    lax.fori_loop(0, CK, _zrow, 0)
    base_row = s * ROWS_PT
    slabs = [(k * CK, CK) for k in range(ROWS_PT // CK)]
    if ROWS_PT % CK:
        slabs.append(((ROWS_PT // CK) * CK, ROWS_PT % CK))
    for (r0, nr) in slabs:
        pltpu.sync_copy(gat0.at[pl.ds(0, nr)],
                        acc_sh.at[pl.ds(base_row + r0, nr)])
    plsc.subcore_barrier()

    cbase = wid * NCH

    def _issue_idx(j, p):
        pkp, valv = pk[p]
        pltpu.async_copy(pk_hbm.at[cbase + j], pkp, sem_idx[p])
        pltpu.async_copy(val_hbm.at[pl.ds((cbase + j) * CK, CK)], valv,
                         sem_idx[p])

    def _wait_idx(p):
        pkp, valv = pk[p]
        pltpu.make_async_copy(pk_hbm.at[0], pkp, sem_idx[p]).wait()
        pltpu.make_async_copy(val_hbm.at[pl.ds(0, CK)], valv,
                              sem_idx[p]).wait()

    def _issue_gat(p):
        pltpu.async_copy(h_hbm.at[pk[p][0].at[0]], gat[p], sem_gat[p])

    def _wait_gat(p):
        pltpu.make_async_copy(h_hbm.at[pk[p][0].at[0]], gat[p],
                              sem_gat[p]).wait()

    def _issue_sc(p):
        pltpu.make_async_copy(gat[p], acc_sh.at[pk[p][0].at[1]],
                              sem_sc[p]).start(add=True)

    def _wait_sc(p):
        pltpu.make_async_copy(gat[p], acc_sh.at[pk[p][0].at[1]],
                              sem_sc[p]).wait()

    def _scale(p):
        valv = pk[p][1]
        gp = gat[p]

        def _grp(g, carry2):
            vv = valv[pl.ds(g * L, L)]
            for e in range(L):
                sv = _splat(vv, e)
                r = g * L + e
                for q in range(D // L):
                    gp[r, pl.ds(q * L, L)] = gp[r, pl.ds(q * L, L)] * sv
            return carry2

        lax.fori_loop(0, CK // L, _grp, 0)

    def _step(j, p, pn, pp):
        _wait_gat(p)

        @pl.when(j >= 1)
        def _():
            _wait_sc(pp)

        @pl.when(j + 2 < NCH)
        def _():
            _issue_idx(j + 2, pp)

        @pl.when(j + 1 < NCH)
        def _():
            _wait_idx(pn)
            _issue_gat(pn)

        _scale(p)
        _issue_sc(p)

    _issue_idx(0, 0)
    _wait_idx(0)
    _issue_gat(0)
    _issue_idx(1, 1)

    def _trip(t, carry):
        j = 3 * t
        _step(j, 0, 1, 2)
        _step(j + 1, 1, 2, 0)
        _step(j + 2, 2, 0, 1)
        return carry

    lax.fori_loop(0, NCH // 3, _trip, 0)
    _wait_sc((NCH - 1) % 3)
    plsc.subcore_barrier()

    for (r0, nr) in slabs:
        pltpu.sync_copy(acc_sh.at[pl.ds(base_row + r0, nr)],
                        gat0.at[pl.ds(0, nr)])
        pltpu.sync_copy(gat0.at[pl.ds(0, nr)],
                        out_hbm.at[pl.ds(c * NPAD + base_row + r0, nr)])


_LN_BLK = 256


def _ln_body(p_ref, g_ref, be_ref, o_ref):
    i = pl.program_id(0)
    h = jnp.maximum(p_ref[0] + p_ref[1], 0.0)
    mu = jnp.mean(h, axis=-1, keepdims=True)
    d = h - mu
    var = jnp.mean(d * d, axis=-1, keepdims=True)
    y = d * lax.rsqrt(var + 1e-5) * g_ref[...] + be_ref[...]
    rows = i * _LN_BLK + lax.broadcasted_iota(jnp.int32, (_LN_BLK, 1), 0)
    o_ref[...] = jnp.where(rows < N, y, 0.0)


def _layernorm(partials, gamma, beta):
    return pl.pallas_call(
        _ln_body,
        grid=(NPAD // _LN_BLK,),
        in_specs=[
            pl.BlockSpec((NC, _LN_BLK, D), lambda i: (0, i, 0)),
            pl.BlockSpec((1, D), lambda i: (0, 0)),
            pl.BlockSpec((1, D), lambda i: (0, 0)),
        ],
        out_specs=pl.BlockSpec((_LN_BLK, D), lambda i: (i, 0)),
        out_shape=jax.ShapeDtypeStruct((NPAD, D), jnp.float32),
    )(partials.reshape(NC, NPAD, D), gamma.reshape(1, D), beta.reshape(1, D))


BPT = B // NT
CKD = 128

@functools.partial(
    pl.kernel,
    out_type=jax.ShapeDtypeStruct((B, D), jnp.float32),
    mesh=_mesh,
    scratch_types=[
        pltpu.VMEM((CKD,), jnp.int32),
        pltpu.VMEM((CKD, D), jnp.float32),
        pltpu.SemaphoreType.DMA,
    ],
)
def _lookup(hn_hbm, x_hbm, out_hbm, idx_v, rows_v, sem):
    c = lax.axis_index("c")
    s = lax.axis_index("s")
    wid = c * NS + s
    base = wid * BPT

    def _chunk(j, carry):
        off = base + j * CKD
        pltpu.sync_copy(x_hbm.at[pl.ds(off, CKD)], idx_v)

        def _grp(g, carry2):
            xx = idx_v[pl.ds(g * L, L)]
            ok = (xx >= 1) & (xx < N + 1)
            idx_v[pl.ds(g * L, L)] = jnp.where(ok, xx - 1, N)
            return carry2

        lax.fori_loop(0, CKD // L, _grp, 0)
        pltpu.async_copy(hn_hbm.at[idx_v], rows_v, sem).wait()
        pltpu.sync_copy(rows_v, out_hbm.at[pl.ds(off, CKD)])
        return carry

    lax.fori_loop(0, BPT // CKD, _chunk, 0)


def kernel(x, embed, W, b, edge_row, edge_col, edge_val, ln_gamma, ln_beta):
    h = _matmul(embed.astype(jnp.float32), W, b)
    pad = EPAD - E
    col_p = jnp.pad(edge_col, (0, pad)).reshape(-1, CK)
    row_p = jnp.pad(edge_row, (0, pad)).reshape(-1, CK)
    val_p = jnp.pad(edge_val, (0, pad))
    pk2 = jnp.stack([col_p, row_p], axis=1)
    partials = _aggregate(h, pk2, val_p)
    hn = _layernorm(partials, ln_gamma, ln_beta)
    final = _lookup(hn, x)
    recon_loss = jnp.zeros((1,), jnp.float32)
    return (final, recon_loss)


# speedup vs baseline: 6.5458x; 1.0455x over previous
import functools

import jax
import jax.numpy as jnp
from jax import lax
from jax.experimental import pallas as pl
from jax.experimental.pallas import tpu as pltpu
from jax.experimental.pallas import tpu_sc as plsc

N = 10000
D = 128
E = 320000
B = 16384

NC = 2
NS = 16
L = 16
NT = NC * NS

NPAD = 10240
CK = 112
NCH = 90
EPT = NCH * CK
EPAD = NT * EPT
ROWS_PT = NPAD // NS

def _splat(vec, e):
    return lax.gather(
        vec,
        jnp.full((L, 1), e, jnp.int32),
        dimension_numbers=lax.GatherDimensionNumbers(
            offset_dims=(), collapsed_slice_dims=(0,), start_index_map=(0,)),
        slice_sizes=(1,),
        mode=lax.GatherScatterMode.PROMISE_IN_BOUNDS,
    )


_mesh = plsc.VectorSubcoreMesh(
    core_axis_name="c", subcore_axis_name="s", num_cores=NC, num_subcores=NS
)


def _mm_body(emb_ref, w_ref, b_ref, out_ref):
    out_ref[...] = (
        jnp.dot(emb_ref[...], w_ref[...], preferred_element_type=jnp.float32)
        + b_ref[...]
    )


def _matmul(embed, W, b):
    return pl.pallas_call(
        _mm_body,
        grid=(5,),
        in_specs=[
            pl.BlockSpec((2000, D), lambda i: (i, 0)),
            pl.BlockSpec((D, D), lambda i: (0, 0)),
            pl.BlockSpec((1, D), lambda i: (0, 0)),
        ],
        out_specs=pl.BlockSpec((2000, D), lambda i: (i, 0)),
        out_shape=jax.ShapeDtypeStruct((N, D), jnp.float32),
    )(embed, W, b.reshape(1, D))


@functools.partial(
    pl.kernel,
    out_type=jax.ShapeDtypeStruct((NC * NPAD, D), jnp.float32),
    mesh=_mesh,
    scratch_types=[
        [pltpu.VMEM((2, CK), jnp.int32),
         pltpu.VMEM((CK,), jnp.float32)],
        [pltpu.VMEM((2, CK), jnp.int32),
         pltpu.VMEM((CK,), jnp.float32)],
        [pltpu.VMEM((2, CK), jnp.int32),
         pltpu.VMEM((CK,), jnp.float32)],
        pltpu.VMEM((CK, D), jnp.float32),
        pltpu.VMEM((CK, D), jnp.float32),
        pltpu.VMEM((CK, D), jnp.float32),
        pltpu.VMEM_SHARED((NPAD, D), jnp.float32),
        pltpu.SemaphoreType.DMA,
        pltpu.SemaphoreType.DMA,
        pltpu.SemaphoreType.DMA,
        pltpu.SemaphoreType.DMA,
        pltpu.SemaphoreType.DMA,
        pltpu.SemaphoreType.DMA,
        pltpu.SemaphoreType.DMA,
        pltpu.SemaphoreType.DMA,
        pltpu.SemaphoreType.DMA,
    ],
)
def _aggregate(h_hbm, pk_hbm, val_hbm, out_hbm,
               pk0, pk1, pk2, gat0, gat1, gat2, acc_sh,
               si0, si1, si2, sg0, sg1, sg2, ss0, ss1, ss2):
    c = lax.axis_index("c")
    s = lax.axis_index("s")
    wid = c * NS + s
    pk = (pk0, pk1, pk2)
    gat = (gat0, gat1, gat2)
    sem_idx = (si0, si1, si2)
    sem_gat = (sg0, sg1, sg2)
    sem_sc = (ss0, ss1, ss2)

    zero = jnp.zeros((L,), jnp.float32)

    def _zrow(i, carry):
        for q in range(D // L):
            gat0[i, pl.ds(q * L, L)] = zero
        return carry

    lax.fori_loop(0, CK, _zrow, 0)
    base_row = s * ROWS_PT
    slabs = [(k * CK, CK) for k in range(ROWS_PT // CK)]
    if ROWS_PT % CK:
        slabs.append(((ROWS_PT // CK) * CK, ROWS_PT % CK))
    for (r0, nr) in slabs:
        pltpu.sync_copy(gat0.at[pl.ds(0, nr)],
                        acc_sh.at[pl.ds(base_row + r0, nr)])
    plsc.subcore_barrier()

    cbase = wid * NCH

    def _issue_idx(j, p):
        pkp, valv = pk[p]
        pltpu.async_copy(pk_hbm.at[cbase + j], pkp, sem_idx[p])
        pltpu.async_copy(val_hbm.at[pl.ds((cbase + j) * CK, CK)], valv,
                         sem_idx[p])

    def _wait_idx(p):
        pkp, valv = pk[p]
        pltpu.make_async_copy(pk_hbm.at[0], pkp, sem_idx[p]).wait()
        pltpu.make_async_copy(val_hbm.at[pl.ds(0, CK)], valv,
                              sem_idx[p]).wait()

    def _issue_gat(p):
        pltpu.async_copy(h_hbm.at[pk[p][0].at[0]], gat[p], sem_gat[p])

    def _wait_gat(p):
        pltpu.make_async_copy(h_hbm.at[pk[p][0].at[0]], gat[p],
                              sem_gat[p]).wait()

    def _issue_sc(p):
        pltpu.make_async_copy(gat[p], acc_sh.at[pk[p][0].at[1]],
                              sem_sc[p]).start(add=True)

    def _wait_sc(p):
        pltpu.make_async_copy(gat[p], acc_sh.at[pk[p][0].at[1]],
                              sem_sc[p]).wait()

    def _scale(p):
        valv = pk[p][1]
        gp = gat[p]

        def _grp(g, carry2):
            vv = valv[pl.ds(g * L, L)]
            for e in range(L):
                sv = _splat(vv, e)
                r = g * L + e
                for q in range(D // L):
                    gp[r, pl.ds(q * L, L)] = gp[r, pl.ds(q * L, L)] * sv
            return carry2

        lax.fori_loop(0, CK // L, _grp, 0)

    def _step(j, p, pn, pp):
        _wait_gat(p)

        @pl.when(j >= 1)
        def _():
            _wait_sc(pp)

        @pl.when(j + 2 < NCH)
        def _():
            _issue_idx(j + 2, pp)

        @pl.when(j + 1 < NCH)
        def _():
            _wait_idx(pn)
            _issue_gat(pn)

        _scale(p)
        _issue_sc(p)

    _issue_idx(0, 0)
    _wait_idx(0)
    _issue_gat(0)
    _issue_idx(1, 1)

    def _trip(t, carry):
        j = 3 * t
        _step(j, 0, 1, 2)
        _step(j + 1, 1, 2, 0)
        _step(j + 2, 2, 0, 1)
        return carry

    lax.fori_loop(0, NCH // 3, _trip, 0)
    _wait_sc((NCH - 1) % 3)
    plsc.subcore_barrier()

    for (r0, nr) in slabs:
        pltpu.sync_copy(acc_sh.at[pl.ds(base_row + r0, nr)],
                        gat0.at[pl.ds(0, nr)])
        pltpu.sync_copy(gat0.at[pl.ds(0, nr)],
                        out_hbm.at[pl.ds(c * NPAD + base_row + r0, nr)])


_LN_BLK = 512


def _ln_body(p_ref, g_ref, be_ref, o_ref):
    i = pl.program_id(0)
    h = jnp.maximum(p_ref[0] + p_ref[1], 0.0)
    mu = jnp.mean(h, axis=-1, keepdims=True)
    d = h - mu
    var = jnp.mean(d * d, axis=-1, keepdims=True)
    y = d * lax.rsqrt(var + 1e-5) * g_ref[...] + be_ref[...]
    rows = i * _LN_BLK + lax.broadcasted_iota(jnp.int32, (_LN_BLK, 1), 0)
    o_ref[...] = jnp.where(rows < N, y, 0.0)


def _layernorm(partials, gamma, beta):
    return pl.pallas_call(
        _ln_body,
        grid=(NPAD // _LN_BLK,),
        in_specs=[
            pl.BlockSpec((NC, _LN_BLK, D), lambda i: (0, i, 0)),
            pl.BlockSpec((1, D), lambda i: (0, 0)),
            pl.BlockSpec((1, D), lambda i: (0, 0)),
        ],
        out_specs=pl.BlockSpec((_LN_BLK, D), lambda i: (i, 0)),
        out_shape=jax.ShapeDtypeStruct((NPAD, D), jnp.float32),
    )(partials.reshape(NC, NPAD, D), gamma.reshape(1, D), beta.reshape(1, D))


BPT = B // NT
CKD = 128

@functools.partial(
    pl.kernel,
    out_type=jax.ShapeDtypeStruct((B, D), jnp.float32),
    mesh=_mesh,
    scratch_types=[
        pltpu.VMEM((CKD,), jnp.int32),
        pltpu.VMEM((CKD, D), jnp.float32),
        pltpu.SemaphoreType.DMA,
    ],
)
def _lookup(hn_hbm, x_hbm, out_hbm, idx_v, rows_v, sem):
    c = lax.axis_index("c")
    s = lax.axis_index("s")
    wid = c * NS + s
    base = wid * BPT

    def _chunk(j, carry):
        off = base + j * CKD
        pltpu.sync_copy(x_hbm.at[pl.ds(off, CKD)], idx_v)

        def _grp(g, carry2):
            xx = idx_v[pl.ds(g * L, L)]
            ok = (xx >= 1) & (xx < N + 1)
            idx_v[pl.ds(g * L, L)] = jnp.where(ok, xx - 1, N)
            return carry2

        lax.fori_loop(0, CKD // L, _grp, 0)
        pltpu.async_copy(hn_hbm.at[idx_v], rows_v, sem).wait()
        pltpu.sync_copy(rows_v, out_hbm.at[pl.ds(off, CKD)])
        return carry

    lax.fori_loop(0, BPT // CKD, _chunk, 0)


def kernel(x, embed, W, b, edge_row, edge_col, edge_val, ln_gamma, ln_beta):
    h = _matmul(embed.astype(jnp.float32), W, b)
    pad = EPAD - E
    col_p = jnp.pad(edge_col, (0, pad)).reshape(-1, CK)
    row_p = jnp.pad(edge_row, (0, pad)).reshape(-1, CK)
    val_p = jnp.pad(edge_val, (0, pad))
    pk2 = jnp.stack([col_p, row_p], axis=1)
    partials = _aggregate(h, pk2, val_p)
    hn = _layernorm(partials, ln_gamma, ln_beta)
    final = _lookup(hn, x)
    recon_loss = jnp.zeros((1,), jnp.float32)
    return (final, recon_loss)
